# baseline (speedup 1.0000x reference)
import functools

import jax
import jax.numpy as jnp
from jax.experimental import pallas as pl

CB = 128
D = 32
BLOCK = 2048
SEQ = 8


def _vq_body(lat_ref, cb_ref, q_ref, p_ref, cbs_ref):
    x = lat_ref[...]
    cb = cb_ref[...]
    s = jnp.sum(x * x, axis=1, keepdims=True)
    n = jnp.sum(cb * cb, axis=1)[None, :]
    mm = jax.lax.dot_general(x, cb, (((1,), (1,)), ((), ())),
                             preferred_element_type=jnp.float32)
    d = (s + n) - 2.0 * mm
    dmin = jnp.min(d, axis=1, keepdims=True)
    lane = jax.lax.broadcasted_iota(jnp.int32, (BLOCK, CB), 1)
    idx = jnp.min(jnp.where(d == dmin, lane, CB), axis=1, keepdims=True)
    oh = (lane == idx).astype(jnp.float32)
    q = jax.lax.dot_general(oh, cb, (((1,), (0,)), ((), ())),
                            preferred_element_type=jnp.float32)
    q_ref[...] = q
    p_ref[...] = q
    cbs_ref[...] = jnp.broadcast_to(cb[None], (BLOCK // SEQ, CB, D))


def kernel(latent, codebook):
    B = latent.shape[0]
    rows = B * SEQ
    lat2 = latent.reshape(rows, D)
    grid = rows // BLOCK
    q, p, cbs = pl.pallas_call(
        _vq_body,
        grid=(grid,),
        in_specs=[
            pl.BlockSpec((BLOCK, D), lambda i: (i, 0)),
            pl.BlockSpec((CB, D), lambda i: (0, 0)),
        ],
        out_specs=[
            pl.BlockSpec((BLOCK, D), lambda i: (i, 0)),
            pl.BlockSpec((BLOCK, D), lambda i: (i, 0)),
            pl.BlockSpec((BLOCK // SEQ, CB, D), lambda i: (i, 0, 0)),
        ],
        out_shape=[
            jax.ShapeDtypeStruct((rows, D), jnp.float32),
            jax.ShapeDtypeStruct((rows, D), jnp.float32),
            jax.ShapeDtypeStruct((B, CB, D), jnp.float32),
        ],
    )(lat2, codebook)
    shape = latent.shape
    return (p.reshape(shape), q.reshape(shape), cbs)


# speedup vs baseline: 1.6102x; 1.6102x over previous
import functools

import jax
import jax.numpy as jnp
from jax.experimental import pallas as pl

CB = 128
D = 32
BLOCK = 2048
SEQ = 8


def _vq_body(lat_ref, cb_ref, cbf_ref, q_ref, p_ref, cbs_ref):
    x = lat_ref[...]
    cb = cb_ref[...]
    s = jnp.sum(x * x, axis=1, keepdims=True)
    n = jnp.sum(cb * cb, axis=1)[None, :]
    mm = jax.lax.dot_general(x, cb, (((1,), (1,)), ((), ())),
                             preferred_element_type=jnp.float32)
    d = (s + n) - 2.0 * mm
    dmin = jnp.min(d, axis=1, keepdims=True)
    lane = jax.lax.broadcasted_iota(jnp.int32, (BLOCK, CB), 1)
    idx = jnp.min(jnp.where(d == dmin, lane, CB), axis=1, keepdims=True)
    oh = (lane == idx).astype(jnp.float32)
    q = jax.lax.dot_general(oh, cb, (((1,), (0,)), ((), ())),
                            preferred_element_type=jnp.float32)
    q_ref[...] = q
    p_ref[...] = q
    cbs_ref[...] = jnp.broadcast_to(cbf_ref[...], (BLOCK // SEQ, CB * D))


def kernel(latent, codebook):
    B = latent.shape[0]
    rows = B * SEQ
    lat2 = latent.reshape(rows, D)
    grid = rows // BLOCK
    q, p, cbs = pl.pallas_call(
        _vq_body,
        grid=(grid,),
        in_specs=[
            pl.BlockSpec((BLOCK, D), lambda i: (i, 0)),
            pl.BlockSpec((CB, D), lambda i: (0, 0)),
            pl.BlockSpec((1, CB * D), lambda i: (0, 0)),
        ],
        out_specs=[
            pl.BlockSpec((BLOCK, D), lambda i: (i, 0)),
            pl.BlockSpec((BLOCK, D), lambda i: (i, 0)),
            pl.BlockSpec((BLOCK // SEQ, CB * D), lambda i: (i, 0)),
        ],
        out_shape=[
            jax.ShapeDtypeStruct((rows, D), jnp.float32),
            jax.ShapeDtypeStruct((rows, D), jnp.float32),
            jax.ShapeDtypeStruct((B, CB * D), jnp.float32),
        ],
    )(lat2, codebook, codebook.reshape(1, CB * D))
    shape = latent.shape
    return (p.reshape(shape), q.reshape(shape), cbs.reshape(B, CB, D))
